# baseline (speedup 1.0000x reference)
import functools

import numpy as np
import jax
import jax.numpy as jnp
from jax import lax
from jax.experimental import pallas as pl
from jax.experimental.pallas import tpu as pltpu
from jax.experimental.pallas import tpu_sc as plsc

VOCAB = 1_000_000
SEQ = 16384
DIM = 64
NC = 2
NS = 16
NW = NC * NS
BPW = SEQ // NW
LANES = 16
W = 16384
NBLK = VOCAB // W // 2
DSHIFT = NBLK * W
GRID = -(-(VOCAB - DSHIFT) // W)
PACKED_ROWS = GRID * W
CB = 2048


def _pos_encoding_np(L: int, d: int) -> np.ndarray:
    pos = np.arange(L, dtype=np.float32)[:, None]
    i = np.arange(d, dtype=np.float32)[None, :]
    angle = pos / np.power(10000.0, 2.0 * i / float(d))
    even = (np.arange(d)[None, :] % 2) == 0
    return np.where(even, np.sin(angle), np.cos(angle)).astype(np.float32)


_POS = _pos_encoding_np(SEQ, DIM)


def _pack_body(a_ref, b_ref, o_ref):
    o_ref[:, 0:DIM] = a_ref[...].T
    o_ref[:, DIM:2 * DIM] = b_ref[...].T


def _transpose_pack(tT):
    return pl.pallas_call(
        _pack_body,
        out_shape=jax.ShapeDtypeStruct((PACKED_ROWS, 2 * DIM), jnp.float32),
        grid=(GRID,),
        in_specs=[
            pl.BlockSpec((DIM, W), lambda i: (0, i)),
            pl.BlockSpec((DIM, W), lambda i: (0, i + NBLK)),
        ],
        out_specs=pl.BlockSpec((W, 2 * DIM), lambda i: (i, 0)),
    )(tT, tT)


_mesh = plsc.VectorSubcoreMesh(core_axis_name="c", subcore_axis_name="s")


@functools.partial(
    pl.kernel,
    mesh=_mesh,
    out_type=jax.ShapeDtypeStruct((SEQ, 2 * DIM), jnp.float32),
    scratch_types=[
        pltpu.VMEM((BPW,), jnp.int32),
        pltpu.VMEM((BPW, 2 * DIM), jnp.float32),
        pltpu.SemaphoreType.DMA,
    ],
    compiler_params=pltpu.CompilerParams(use_tc_tiling_on_sc=False),
)
def _gather(x_hbm, tp_hbm, out_hbm, idx_v, rows_v, gsem):
    wid = lax.axis_index("s") * NC + lax.axis_index("c")
    base = wid * BPW

    pltpu.sync_copy(x_hbm.at[pl.ds(base, BPW)], idx_v)

    for g in range(BPW // LANES):
        sl = pl.ds(g * LANES, LANES)
        v = idx_v[sl]
        hi = 1 + lax.shift_right_arithmetic(v - DSHIFT, 31)
        idx_v[sl] = v - hi * DSHIFT

    gathers = []
    for j in range(BPW // 128):
        gathers.append(
            pltpu.async_copy(
                tp_hbm.at[idx_v.at[pl.ds(j * 128, 128)]],
                rows_v.at[pl.ds(j * 128, 128)],
                gsem,
            )
        )
    for g in gathers:
        g.wait()

    pltpu.sync_copy(rows_v, out_hbm.at[pl.ds(base, BPW)])


def _combine_body(rows_ref, sel_ref, pos_ref, o_ref):
    rows = rows_ref[...].astype(jnp.float32)
    sel = sel_ref[...]
    picked = sel * rows[:, DIM:2 * DIM] + (1.0 - sel) * rows[:, 0:DIM]
    o_ref[...] = picked + pos_ref[...]


def _combine(rows, sel, pos):
    return pl.pallas_call(
        _combine_body,
        out_shape=jax.ShapeDtypeStruct((SEQ, DIM), jnp.float32),
        grid=(SEQ // CB,),
        in_specs=[
            pl.BlockSpec((CB, 2 * DIM), lambda i: (i, 0)),
            pl.BlockSpec((CB, 1), lambda i: (i, 0)),
            pl.BlockSpec((CB, DIM), lambda i: (i, 0)),
        ],
        out_specs=pl.BlockSpec((CB, DIM), lambda i: (i, 0)),
    )(rows, sel, pos)


def kernel(x, table):
    xi = x.astype(jnp.int32)
    pos = jnp.asarray(_POS)
    packed = _transpose_pack(table.T)
    rows = _gather(xi, packed)
    sel = (xi >= DSHIFT).astype(jnp.float32)[:, None]
    return _combine(rows, sel, pos)


# speedup vs baseline: 1.3922x; 1.3922x over previous
import functools

import numpy as np
import jax
import jax.numpy as jnp
from jax import lax
from jax.experimental import pallas as pl
from jax.experimental.pallas import tpu as pltpu
from jax.experimental.pallas import tpu_sc as plsc

VOCAB = 1_000_000
SEQ = 16384
DIM = 64
NC = 2
NS = 16
NW = NC * NS
BPW = SEQ // NW
LANES = 16
W = 16384
NBLK = VOCAB // W // 2
DSHIFT = NBLK * W
GRID = -(-(VOCAB - DSHIFT) // W)
PACKED_ROWS = GRID * W // 2
CB = 2048


def _pos_encoding_np(L: int, d: int) -> np.ndarray:
    pos = np.arange(L, dtype=np.float32)[:, None]
    i = np.arange(d, dtype=np.float32)[None, :]
    angle = pos / np.power(10000.0, 2.0 * i / float(d))
    even = (np.arange(d)[None, :] % 2) == 0
    return np.where(even, np.sin(angle), np.cos(angle)).astype(np.float32)


_POS = _pos_encoding_np(SEQ, DIM)


def _pack_body(a_ref, b_ref, o_ref):
    a16 = a_ref[...].T.astype(jnp.bfloat16)
    b16 = b_ref[...].T.astype(jnp.bfloat16)
    o_ref[:, 0:DIM] = pltpu.bitcast(a16, jnp.float32)
    o_ref[:, DIM:2 * DIM] = pltpu.bitcast(b16, jnp.float32)


def _transpose_pack(tT):
    return pl.pallas_call(
        _pack_body,
        out_shape=jax.ShapeDtypeStruct((PACKED_ROWS, 2 * DIM), jnp.float32),
        grid=(GRID,),
        in_specs=[
            pl.BlockSpec((DIM, W), lambda i: (0, i)),
            pl.BlockSpec((DIM, W), lambda i: (0, i + NBLK)),
        ],
        out_specs=pl.BlockSpec((W // 2, 2 * DIM), lambda i: (i, 0)),
    )(tT, tT)


_mesh = plsc.VectorSubcoreMesh(core_axis_name="c", subcore_axis_name="s")


@functools.partial(
    pl.kernel,
    mesh=_mesh,
    out_type=jax.ShapeDtypeStruct((SEQ, 2 * DIM), jnp.float32),
    scratch_types=[
        pltpu.VMEM((BPW,), jnp.int32),
        pltpu.VMEM((BPW, 2 * DIM), jnp.float32),
        pltpu.SemaphoreType.DMA,
    ],
    compiler_params=pltpu.CompilerParams(use_tc_tiling_on_sc=False),
)
def _gather(x_hbm, tp_hbm, out_hbm, idx_v, rows_v, gsem):
    wid = lax.axis_index("s") * NC + lax.axis_index("c")
    base = wid * BPW

    pltpu.sync_copy(x_hbm.at[pl.ds(base, BPW)], idx_v)

    for g in range(BPW // LANES):
        sl = pl.ds(g * LANES, LANES)
        v = idx_v[sl]
        hi = 1 + lax.shift_right_arithmetic(v - DSHIFT, 31)
        idx_v[sl] = lax.shift_right_logical(v - hi * DSHIFT, 1)

    gathers = []
    for j in range(BPW // 128):
        gathers.append(
            pltpu.async_copy(
                tp_hbm.at[idx_v.at[pl.ds(j * 128, 128)]],
                rows_v.at[pl.ds(j * 128, 128)],
                gsem,
            )
        )
    for g in gathers:
        g.wait()

    pltpu.sync_copy(rows_v, out_hbm.at[pl.ds(base, BPW)])


def _combine_body(rows_ref, sel_ref, par_ref, pos_ref, o_ref):
    w = lax.bitcast_convert_type(rows_ref[...], jnp.int32)
    sel = sel_ref[...] != 0
    par = par_ref[...] != 0
    wsel = jnp.where(sel, w[:, DIM:2 * DIM], w[:, 0:DIM])
    lo = lax.shift_left(wsel, 16)
    hi = lax.bitwise_and(wsel, jnp.int32(-65536))
    v = lax.bitcast_convert_type(jnp.where(par, hi, lo), jnp.float32)
    o_ref[...] = v + pos_ref[...]


def _combine(rows, sel, par, pos):
    return pl.pallas_call(
        _combine_body,
        out_shape=jax.ShapeDtypeStruct((SEQ, DIM), jnp.float32),
        grid=(SEQ // CB,),
        in_specs=[
            pl.BlockSpec((CB, 2 * DIM), lambda i: (i, 0)),
            pl.BlockSpec((CB, 1), lambda i: (i, 0)),
            pl.BlockSpec((CB, 1), lambda i: (i, 0)),
            pl.BlockSpec((CB, DIM), lambda i: (i, 0)),
        ],
        out_specs=pl.BlockSpec((CB, DIM), lambda i: (i, 0)),
    )(rows, sel, par, pos)


def kernel(x, table):
    xi = x.astype(jnp.int32)
    pos = jnp.asarray(_POS)
    packed = _transpose_pack(table.T)
    rows = _gather(xi, packed)
    sel = (xi >= DSHIFT).astype(jnp.int32)[:, None]
    par = (xi & 1)[:, None]
    return _combine(rows, sel, par, pos)


# speedup vs baseline: 1.4199x; 1.0199x over previous
import functools

import numpy as np
import jax
import jax.numpy as jnp
from jax import lax
from jax.experimental import pallas as pl
from jax.experimental.pallas import tpu as pltpu
from jax.experimental.pallas import tpu_sc as plsc

VOCAB = 1_000_000
SEQ = 16384
DIM = 64
NC = 2
NS = 16
NW = NC * NS
BPW = SEQ // NW
LANES = 16
W = 32768
NBLK = VOCAB // W // 2
DSHIFT = NBLK * W
GRID = -(-(VOCAB - DSHIFT) // W)
PACKED_ROWS = GRID * W // 2
CB = 2048


def _pos_encoding_np(L: int, d: int) -> np.ndarray:
    pos = np.arange(L, dtype=np.float32)[:, None]
    i = np.arange(d, dtype=np.float32)[None, :]
    angle = pos / np.power(10000.0, 2.0 * i / float(d))
    even = (np.arange(d)[None, :] % 2) == 0
    return np.where(even, np.sin(angle), np.cos(angle)).astype(np.float32)


_POS = _pos_encoding_np(SEQ, DIM)


def _pack_body(a_ref, b_ref, o_ref):
    a16 = a_ref[...].T.astype(jnp.bfloat16)
    b16 = b_ref[...].T.astype(jnp.bfloat16)
    o_ref[:, 0:DIM] = pltpu.bitcast(a16, jnp.float32)
    o_ref[:, DIM:2 * DIM] = pltpu.bitcast(b16, jnp.float32)


def _transpose_pack(tT):
    return pl.pallas_call(
        _pack_body,
        out_shape=jax.ShapeDtypeStruct((PACKED_ROWS, 2 * DIM), jnp.float32),
        grid=(GRID,),
        in_specs=[
            pl.BlockSpec((DIM, W), lambda i: (0, i)),
            pl.BlockSpec((DIM, W), lambda i: (0, i + NBLK)),
        ],
        out_specs=pl.BlockSpec((W // 2, 2 * DIM), lambda i: (i, 0)),
    )(tT, tT)


_mesh = plsc.VectorSubcoreMesh(core_axis_name="c", subcore_axis_name="s")


@functools.partial(
    pl.kernel,
    mesh=_mesh,
    out_type=jax.ShapeDtypeStruct((SEQ, 2 * DIM), jnp.float32),
    scratch_types=[
        pltpu.VMEM((BPW,), jnp.int32),
        pltpu.VMEM((BPW, 2 * DIM), jnp.float32),
        pltpu.SemaphoreType.DMA,
    ],
    compiler_params=pltpu.CompilerParams(use_tc_tiling_on_sc=False),
)
def _gather(x_hbm, tp_hbm, out_hbm, idx_v, rows_v, gsem):
    wid = lax.axis_index("s") * NC + lax.axis_index("c")
    base = wid * BPW

    pltpu.sync_copy(x_hbm.at[pl.ds(base, BPW)], idx_v)

    for g in range(BPW // LANES):
        sl = pl.ds(g * LANES, LANES)
        v = idx_v[sl]
        hi = 1 + lax.shift_right_arithmetic(v - DSHIFT, 31)
        idx_v[sl] = lax.shift_right_logical(v - hi * DSHIFT, 1)

    gathers = []
    for j in range(BPW // 128):
        gathers.append(
            pltpu.async_copy(
                tp_hbm.at[idx_v.at[pl.ds(j * 128, 128)]],
                rows_v.at[pl.ds(j * 128, 128)],
                gsem,
            )
        )
    for g in gathers:
        g.wait()

    pltpu.sync_copy(rows_v, out_hbm.at[pl.ds(base, BPW)])


def _combine_body(rows_ref, sel_ref, par_ref, pos_ref, o_ref):
    w = lax.bitcast_convert_type(rows_ref[...], jnp.int32)
    sel = sel_ref[...] != 0
    par = par_ref[...] != 0
    wsel = jnp.where(sel, w[:, DIM:2 * DIM], w[:, 0:DIM])
    lo = lax.shift_left(wsel, 16)
    hi = lax.bitwise_and(wsel, jnp.int32(-65536))
    v = lax.bitcast_convert_type(jnp.where(par, hi, lo), jnp.float32)
    o_ref[...] = v + pos_ref[...]


def _combine(rows, sel, par, pos):
    return pl.pallas_call(
        _combine_body,
        out_shape=jax.ShapeDtypeStruct((SEQ, DIM), jnp.float32),
        grid=(SEQ // CB,),
        in_specs=[
            pl.BlockSpec((CB, 2 * DIM), lambda i: (i, 0)),
            pl.BlockSpec((CB, 1), lambda i: (i, 0)),
            pl.BlockSpec((CB, 1), lambda i: (i, 0)),
            pl.BlockSpec((CB, DIM), lambda i: (i, 0)),
        ],
        out_specs=pl.BlockSpec((CB, DIM), lambda i: (i, 0)),
    )(rows, sel, par, pos)


def kernel(x, table):
    xi = x.astype(jnp.int32)
    pos = jnp.asarray(_POS)
    packed = _transpose_pack(table.T)
    rows = _gather(xi, packed)
    sel = (xi >= DSHIFT).astype(jnp.int32)[:, None]
    par = (xi & 1)[:, None]
    return _combine(rows, sel, par, pos)
